# baseline (speedup 1.0000x reference)
import functools

import jax
import jax.numpy as jnp
from jax import lax
from jax.experimental import pallas as pl
from jax.experimental.pallas import tpu as pltpu
from jax.experimental.pallas import tpu_sc as plsc

N = 10000
D = 128
DH = D // 2
E = 320000
NP = 10240
NC, NS = 2, 16
NW = NC * NS
K = 128
C = 79
EP = NW * C * K
C2 = 2 * C
SHARE = NP // NS

_mesh = lambda: plsc.VectorSubcoreMesh(
    core_axis_name="c", subcore_axis_name="s", num_cores=NC, num_subcores=NS)


def _deg_kernel(dst3, ones, zeros):
    @functools.partial(
        pl.kernel,
        out_type=jax.ShapeDtypeStruct((NC, NP, 16), jnp.float32),
        mesh=_mesh(),
        compiler_params=pltpu.CompilerParams(use_tc_tiling_on_sc=False),
        scratch_types=[
            pltpu.VMEM_SHARED((NP, 16), jnp.float32),
            pltpu.VMEM((C, K), jnp.int32),
            pltpu.VMEM((K, 16), jnp.float32),
            pltpu.SemaphoreType.DMA,
        ],
    )
    def run(dst_hbm, ones_hbm, zero_hbm, deg_out, deg_sh, idx_v, ones_v, sem):
        cid = lax.axis_index("c")
        sid = lax.axis_index("s")
        pltpu.sync_copy(zero_hbm, deg_sh.at[pl.ds(sid * SHARE, SHARE)])
        pltpu.sync_copy(dst_hbm.at[sid, pl.ds(cid * C, C)], idx_v)
        pltpu.sync_copy(ones_hbm, ones_v)
        plsc.subcore_barrier()
        for j0 in range(0, C, 16):
            descs = [
                pltpu.async_copy(ones_v, deg_sh.at[idx_v.at[j]], sem, add=True)
                for j in range(j0, min(j0 + 16, C))
            ]
            for d in descs:
                d.wait()
        plsc.subcore_barrier()
        pltpu.sync_copy(deg_sh.at[pl.ds(sid * SHARE, SHARE)],
                        deg_out.at[cid, pl.ds(sid * SHARE, SHARE)])

    return run(dst3, ones, zeros)


def _prep_body(x_ref, w_ref, deg_ref, y_ref, st_ref):
    xw = jnp.dot(x_ref[...], w_ref[...], preferred_element_type=jnp.float32)
    t = jnp.min(deg_ref[0], axis=1, keepdims=True) + \
        jnp.min(deg_ref[1], axis=1, keepdims=True) + 1.0
    dinv = lax.rsqrt(t)
    y_ref[...] = xw * dinv
    st_ref[...] = xw * (dinv * dinv)


def _prep_kernel(x_pad, W, deg2):
    blk = 1024
    grid = NP // blk
    return pl.pallas_call(
        _prep_body,
        grid=(grid,),
        in_specs=[
            pl.BlockSpec((blk, D), lambda i: (i, 0)),
            pl.BlockSpec((D, D), lambda i: (0, 0)),
            pl.BlockSpec((NC, blk, 16), lambda i: (0, i, 0)),
        ],
        out_specs=[
            pl.BlockSpec((blk, D), lambda i: (i, 0)),
            pl.BlockSpec((blk, D), lambda i: (i, 0)),
        ],
        out_shape=[
            jax.ShapeDtypeStruct((NP, D), jnp.float32),
            jax.ShapeDtypeStruct((NP, D), jnp.float32),
        ],
    )(x_pad, W, deg2)


NB = 6
NG = 5


def _scatter_kernel(y2, src3, dst3, zeros):
    @functools.partial(
        pl.kernel,
        out_type=jax.ShapeDtypeStruct((NC, NP, DH), jnp.float32),
        mesh=_mesh(),
        compiler_params=pltpu.CompilerParams(use_tc_tiling_on_sc=False),
        scratch_types=(
            [pltpu.VMEM_SHARED((NP, DH), jnp.float32),
             pltpu.VMEM((C2, K), jnp.int32),
             pltpu.VMEM((C2, K), jnp.int32)]
            + [pltpu.VMEM((K, DH), jnp.float32)] * NB
            + [pltpu.SemaphoreType.DMA] * (2 * NB)
        ),
    )
    def run(y_hbm, src_hbm, dst_hbm, zero_hbm, acc_out, acc_sh,
            src_v, dst_v, *bufs_and_sems):
        rows = bufs_and_sems[:NB]
        gsem = bufs_and_sems[NB:2 * NB]
        ssem = bufs_and_sems[2 * NB:]
        cid = lax.axis_index("c")
        sid = lax.axis_index("s")
        pltpu.sync_copy(zero_hbm, acc_sh.at[pl.ds(sid * SHARE, SHARE)])
        pltpu.sync_copy(src_hbm.at[sid], src_v)
        pltpu.sync_copy(dst_hbm.at[sid], dst_v)
        plsc.subcore_barrier()

        def _xform(j):
            for cc in range(K // 16):
                sl = pl.ds(cc * 16, 16)
                src_v[j, sl] = src_v[j, sl] * 2 + cid

        gd = [None] * C2
        sd = [None] * C2
        for j in range(NG):
            _xform(j)
            gd[j] = pltpu.async_copy(y_hbm.at[src_v.at[j]], rows[j % NB],
                                     gsem[j % NB])
        for j in range(C2):
            b = j % NB
            if j - (NB - NG) >= 0:
                sd[j - (NB - NG)].wait()
            gd[j].wait()
            sd[j] = pltpu.async_copy(rows[b], acc_sh.at[dst_v.at[j]],
                                     ssem[b], add=True)
            jn = j + NG
            if jn < C2:
                _xform(jn)
                gd[jn] = pltpu.async_copy(y_hbm.at[src_v.at[jn]],
                                          rows[jn % NB], gsem[jn % NB])
        for j in range(C2 - (NB - NG), C2):
            if j >= 0:
                sd[j].wait()
        plsc.subcore_barrier()
        pltpu.sync_copy(acc_sh.at[pl.ds(sid * SHARE, SHARE)],
                        acc_out.at[cid, pl.ds(sid * SHARE, SHARE)])

    return run(y2, src3, dst3, zeros)


def _final_body(acc_ref, deg_ref, st_ref, b_ref, wh_ref, bh_ref, out_ref):
    t = jnp.min(deg_ref[0], axis=1, keepdims=True) + \
        jnp.min(deg_ref[1], axis=1, keepdims=True) + 1.0
    dinv = lax.rsqrt(t)
    a = jnp.concatenate([acc_ref[0], acc_ref[1]], axis=1)
    h = jnp.maximum(a * dinv + st_ref[...] + b_ref[...], 0.0)
    out_ref[...] = jnp.dot(h, wh_ref[...],
                           preferred_element_type=jnp.float32) + bh_ref[...]


def _final_kernel(acc, deg2, selfterm, b2, W_head, bh2):
    blk = 1024
    grid = NP // blk
    return pl.pallas_call(
        _final_body,
        grid=(grid,),
        in_specs=[
            pl.BlockSpec((NC, blk, DH), lambda i: (0, i, 0)),
            pl.BlockSpec((NC, blk, 16), lambda i: (0, i, 0)),
            pl.BlockSpec((blk, D), lambda i: (i, 0)),
            pl.BlockSpec((1, D), lambda i: (0, 0)),
            pl.BlockSpec((D, D), lambda i: (0, 0)),
            pl.BlockSpec((1, D), lambda i: (0, 0)),
        ],
        out_specs=pl.BlockSpec((blk, D), lambda i: (i, 0)),
        out_shape=jax.ShapeDtypeStruct((N, D), jnp.float32),
    )(acc, deg2, selfterm, b2, W_head, bh2)


def kernel(x, edge_index, W, b, W_head, b_head):
    ei = edge_index.astype(jnp.int32)
    n_pad = EP - E
    src = jnp.concatenate([ei[0], jnp.full((n_pad,), N, jnp.int32)])
    dst = jnp.concatenate([ei[1], jnp.full((n_pad,), NP - 1, jnp.int32)])
    src3 = src.reshape(NS, C2, K)
    dst3 = dst.reshape(NS, C2, K)

    ones16 = jnp.ones((K, 16), jnp.float32)
    zeros16 = jnp.zeros((SHARE, 16), jnp.float32)
    zerosD = jnp.zeros((SHARE, DH), jnp.float32)

    deg2 = _deg_kernel(dst3, ones16, zeros16)
    y, selfterm = _prep_kernel(x, W, deg2)
    y2 = y.reshape(2 * NP, DH)
    acc = _scatter_kernel(y2, src3, dst3, zerosD)
    out = _final_kernel(acc, deg2, selfterm,
                        b.reshape(1, D), W_head, b_head.reshape(1, D))
    return out


# speedup vs baseline: 1.0091x; 1.0091x over previous
import functools

import jax
import jax.numpy as jnp
from jax import lax
from jax.experimental import pallas as pl
from jax.experimental.pallas import tpu as pltpu
from jax.experimental.pallas import tpu_sc as plsc

N = 10000
D = 128
DH = D // 2
E = 320000
NP = 10240
NC, NS = 2, 16
NW = NC * NS
K = 64
C = 158
EP = NW * C * K
SHARE = NP // NS

_mesh = lambda: plsc.VectorSubcoreMesh(
    core_axis_name="c", subcore_axis_name="s", num_cores=NC, num_subcores=NS)


def _deg_kernel(dst3, ones, zeros):
    @functools.partial(
        pl.kernel,
        out_type=jax.ShapeDtypeStruct((NC, NP, 16), jnp.float32),
        mesh=_mesh(),
        compiler_params=pltpu.CompilerParams(use_tc_tiling_on_sc=False),
        scratch_types=[
            pltpu.VMEM_SHARED((NP, 16), jnp.float32),
            pltpu.VMEM((C, K), jnp.int32),
            pltpu.VMEM((K, 16), jnp.float32),
            pltpu.SemaphoreType.DMA,
        ],
    )
    def run(dst_hbm, ones_hbm, zero_hbm, deg_out, deg_sh, idx_v, ones_v, sem):
        cid = lax.axis_index("c")
        sid = lax.axis_index("s")
        wid = cid * NS + sid
        pltpu.sync_copy(zero_hbm, deg_sh.at[pl.ds(sid * SHARE, SHARE)])
        pltpu.sync_copy(dst_hbm.at[wid], idx_v)
        pltpu.sync_copy(ones_hbm, ones_v)
        plsc.subcore_barrier()
        for j0 in range(0, C, 16):
            descs = [
                pltpu.async_copy(ones_v, deg_sh.at[idx_v.at[j]], sem, add=True)
                for j in range(j0, min(j0 + 16, C))
            ]
            for d in descs:
                d.wait()
        plsc.subcore_barrier()
        pltpu.sync_copy(deg_sh.at[pl.ds(sid * SHARE, SHARE)],
                        deg_out.at[cid, pl.ds(sid * SHARE, SHARE)])

    return run(dst3, ones, zeros)


def _prep_body(x_ref, w_ref, deg_ref, y_ref, st_ref):
    xw = jnp.dot(x_ref[...], w_ref[...], preferred_element_type=jnp.float32)
    t = jnp.min(deg_ref[0], axis=1, keepdims=True) + \
        jnp.min(deg_ref[1], axis=1, keepdims=True) + 1.0
    dinv = lax.rsqrt(t)
    y_ref[...] = xw * dinv
    st_ref[...] = xw * (dinv * dinv)


def _prep_kernel(x_pad, W, deg2):
    blk = 1024
    grid = NP // blk
    return pl.pallas_call(
        _prep_body,
        grid=(grid,),
        in_specs=[
            pl.BlockSpec((blk, D), lambda i: (i, 0)),
            pl.BlockSpec((D, D), lambda i: (0, 0)),
            pl.BlockSpec((NC, blk, 16), lambda i: (0, i, 0)),
        ],
        out_specs=[
            pl.BlockSpec((blk, D), lambda i: (i, 0)),
            pl.BlockSpec((blk, D), lambda i: (i, 0)),
        ],
        out_shape=[
            jax.ShapeDtypeStruct((NP, D), jnp.float32),
            jax.ShapeDtypeStruct((NP, D), jnp.float32),
        ],
    )(x_pad, W, deg2)


NB = 3
NG = 2


def _scatter_kernel(y, src3, dst3, zeros):
    @functools.partial(
        pl.kernel,
        out_type=jax.ShapeDtypeStruct((NC, NP, D), jnp.float32),
        mesh=_mesh(),
        compiler_params=pltpu.CompilerParams(use_tc_tiling_on_sc=False),
        scratch_types=(
            [pltpu.VMEM_SHARED((NP, D), jnp.float32),
             pltpu.VMEM((C, K), jnp.int32),
             pltpu.VMEM((C, K), jnp.int32)]
            + [pltpu.VMEM((K, D), jnp.float32)] * NB
            + [pltpu.SemaphoreType.DMA] * (2 * NB)
        ),
    )
    def run(y_hbm, src_hbm, dst_hbm, zero_hbm, acc_out, acc_sh,
            src_v, dst_v, *bufs_and_sems):
        rows = bufs_and_sems[:NB]
        gsem = bufs_and_sems[NB:2 * NB]
        ssem = bufs_and_sems[2 * NB:]
        cid = lax.axis_index("c")
        sid = lax.axis_index("s")
        wid = cid * NS + sid
        pltpu.sync_copy(zero_hbm, acc_sh.at[pl.ds(sid * SHARE, SHARE)])
        pltpu.sync_copy(src_hbm.at[wid], src_v)
        pltpu.sync_copy(dst_hbm.at[wid], dst_v)
        plsc.subcore_barrier()

        gd = [None] * C
        sd = [None] * C
        for j in range(NG):
            gd[j] = pltpu.async_copy(y_hbm.at[src_v.at[j]], rows[j % NB],
                                     gsem[j % NB])
        for j in range(C):
            b = j % NB
            if j - (NB - NG) >= 0:
                sd[j - (NB - NG)].wait()
            gd[j].wait()
            sd[j] = pltpu.async_copy(rows[b], acc_sh.at[dst_v.at[j]],
                                     ssem[b], add=True)
            jn = j + NG
            if jn < C:
                gd[jn] = pltpu.async_copy(y_hbm.at[src_v.at[jn]],
                                          rows[jn % NB], gsem[jn % NB])
        for j in range(C - (NB - NG), C):
            if j >= 0:
                sd[j].wait()
        plsc.subcore_barrier()
        pltpu.sync_copy(acc_sh.at[pl.ds(sid * SHARE, SHARE)],
                        acc_out.at[cid, pl.ds(sid * SHARE, SHARE)])

    return run(y, src3, dst3, zeros)


def _final_body(acc_ref, deg_ref, st_ref, b_ref, wh_ref, bh_ref, out_ref):
    t = jnp.min(deg_ref[0], axis=1, keepdims=True) + \
        jnp.min(deg_ref[1], axis=1, keepdims=True) + 1.0
    dinv = lax.rsqrt(t)
    a = acc_ref[0] + acc_ref[1]
    h = jnp.maximum(a * dinv + st_ref[...] + b_ref[...], 0.0)
    out_ref[...] = jnp.dot(h, wh_ref[...],
                           preferred_element_type=jnp.float32) + bh_ref[...]


def _final_kernel(acc, deg2, selfterm, b2, W_head, bh2):
    blk = 1024
    grid = NP // blk
    return pl.pallas_call(
        _final_body,
        grid=(grid,),
        in_specs=[
            pl.BlockSpec((NC, blk, D), lambda i: (0, i, 0)),
            pl.BlockSpec((NC, blk, 16), lambda i: (0, i, 0)),
            pl.BlockSpec((blk, D), lambda i: (i, 0)),
            pl.BlockSpec((1, D), lambda i: (0, 0)),
            pl.BlockSpec((D, D), lambda i: (0, 0)),
            pl.BlockSpec((1, D), lambda i: (0, 0)),
        ],
        out_specs=pl.BlockSpec((blk, D), lambda i: (i, 0)),
        out_shape=jax.ShapeDtypeStruct((N, D), jnp.float32),
    )(acc, deg2, selfterm, b2, W_head, bh2)


def kernel(x, edge_index, W, b, W_head, b_head):
    ei = edge_index.astype(jnp.int32)
    n_pad = EP - E
    src = jnp.concatenate([ei[0], jnp.full((n_pad,), N, jnp.int32)])
    dst = jnp.concatenate([ei[1], jnp.full((n_pad,), NP - 1, jnp.int32)])
    src3 = src.reshape(NW, C, K)
    dst3 = dst.reshape(NW, C, K)

    ones16 = jnp.ones((K, 16), jnp.float32)
    zeros16 = jnp.zeros((SHARE, 16), jnp.float32)
    zerosD = jnp.zeros((SHARE, D), jnp.float32)

    deg2 = _deg_kernel(dst3, ones16, zeros16)
    y, selfterm = _prep_kernel(x, W, deg2)
    acc = _scatter_kernel(y, src3, dst3, zerosD)
    out = _final_kernel(acc, deg2, selfterm,
                        b.reshape(1, D), W_head, b_head.reshape(1, D))
    return out
